# Initial kernel scaffold; baseline (speedup 1.0000x reference)
import functools

import jax
import jax.numpy as jnp
from jax import lax
from jax.experimental import pallas as pl
from jax.experimental.pallas import tpu as pltpu
from jax.experimental.pallas import tpu_sc as plsc

STRIDE = 2
FSIZE = 3
HO = WO = 55
HI = WI = 111
CI = 64
CO = 128
P = FSIZE * FSIZE * CI
ROWW = WI * CI

NTILES = 16
PAIRS = HO * WO * CO
ROWS_PER_TILE = 190
PAIRS_PAD = NTILES * ROWS_PER_TILE * 128
NOUT = HI * WI * CI
NOUT_PAD = 788608
OUT_CHUNK = NOUT_PAD // NTILES


def _tc_argmax_body(x_ref, w_ref, out_ref):
    ho = pl.program_id(0)
    neg_inf = jnp.float32(-jnp.inf)
    maxv = jnp.full((WO, CO), neg_inf, dtype=jnp.float32)
    maxp = jnp.zeros((WO, CO), dtype=jnp.int32)
    for dh in range(FSIZE):
        row = x_ref[2 * ho + dh]
        pairs = row[0:110].reshape(WO, 2, CI)
        e = pairs[:, 0, :]
        o = pairs[:, 1, :]
        e2 = jnp.concatenate([e[1:], row[110:111]], axis=0)
        slabs = (e, o, e2)
        for dw in range(FSIZE):
            s = slabs[dw]
            for ci in range(CI):
                p = (dh * FSIZE + dw) * CI + ci
                prod = s[:, ci:ci + 1] * w_ref[p:p + 1, :]
                upd = prod > maxv
                maxv = jnp.where(upd, prod, maxv)
                maxp = jnp.where(upd, p, maxp)
    ph = maxp // (FSIZE * CI)
    pw = (maxp // CI) % FSIZE
    pc = maxp % CI
    wo = lax.broadcasted_iota(jnp.int32, (WO, CO), 0)
    gh = ph + 2 * ho
    gw = pw + 2 * wo
    out_ref[0] = gh * ROWW + gw * CI + pc


def _tc_argmax(x, w):
    return pl.pallas_call(
        _tc_argmax_body,
        grid=(HO,),
        in_specs=[
            pl.BlockSpec((HI, WI, CI), lambda i: (0, 0, 0)),
            pl.BlockSpec((P, CO), lambda i: (0, 0)),
        ],
        out_specs=pl.BlockSpec((1, WO, CO), lambda i: (i, 0, 0)),
        out_shape=jax.ShapeDtypeStruct((HO, WO, CO), jnp.int32),
    )(x, w)


def _sc_scatter_body(idx_hbm, val_hbm, zeros_hbm, out_hbm, idx_v, val_v, acc):
    c = lax.axis_index("c")
    s = lax.axis_index("s")

    @pl.when(c == 0)
    def _():
        pltpu.sync_copy(zeros_hbm.at[pl.ds(s * OUT_CHUNK, OUT_CHUNK)],
                        acc.at[pl.ds(s * OUT_CHUNK, OUT_CHUNK)])
        pltpu.sync_copy(idx_hbm.at[pl.ds(s * ROWS_PER_TILE, ROWS_PER_TILE)],
                        idx_v)
        pltpu.sync_copy(val_hbm.at[pl.ds(s * ROWS_PER_TILE, ROWS_PER_TILE)],
                        val_v)
        plsc.subcore_barrier()

        def body(j, carry):
            pltpu.sync_copy(val_v.at[j], acc.at[idx_v.at[j]], add=True)
            return carry

        lax.fori_loop(0, ROWS_PER_TILE, body, jnp.int32(0))
        plsc.subcore_barrier()
        pltpu.sync_copy(acc.at[pl.ds(s * OUT_CHUNK, OUT_CHUNK)],
                        out_hbm.at[pl.ds(s * OUT_CHUNK, OUT_CHUNK)])


@functools.partial(
    pl.kernel,
    mesh=plsc.VectorSubcoreMesh(core_axis_name="c", subcore_axis_name="s"),
    out_type=jax.ShapeDtypeStruct((NOUT_PAD,), jnp.float32),
    scratch_types=[
        pltpu.VMEM((ROWS_PER_TILE, 128), jnp.int32),
        pltpu.VMEM((ROWS_PER_TILE, 128), jnp.float32),
        pltpu.VMEM_SHARED((NOUT_PAD,), jnp.float32),
    ],
)
def _sc_scatter(idx_hbm, val_hbm, zeros_hbm, out_hbm, idx_v, val_v, acc):
    _sc_scatter_body(idx_hbm, val_hbm, zeros_hbm, out_hbm, idx_v, val_v, acc)


def kernel(inputs, layer_output, layer_weights):
    x = layer_output[0]
    w = layer_weights.reshape(P, CO)
    gidx = _tc_argmax(x, w)
    idx_flat = gidx.reshape(-1)
    val_flat = inputs.reshape(-1)
    pad = PAIRS_PAD - PAIRS
    idx2d = jnp.concatenate(
        [idx_flat, jnp.zeros((pad,), jnp.int32)]).reshape(-1, 128)
    val2d = jnp.concatenate(
        [val_flat, jnp.zeros((pad,), jnp.float32)]).reshape(-1, 128)
    zeros = jnp.zeros((NOUT_PAD,), jnp.float32)
    out = _sc_scatter(idx2d, val2d, zeros)
    return out[:NOUT].reshape(1, HI, WI, CI)


# speedup vs baseline: 4.1910x; 4.1910x over previous
import functools

import jax
import jax.numpy as jnp
from jax import lax
from jax.experimental import pallas as pl
from jax.experimental.pallas import tpu as pltpu
from jax.experimental.pallas import tpu_sc as plsc

STRIDE = 2
FSIZE = 3
HO = WO = 55
HI = WI = 111
CI = 64
CO = 128
P = FSIZE * FSIZE * CI
ROWW = WI * CI

NTILES = 16
PAIRS = HO * WO * CO
ROWS_PER_TILE = 192
PAIRS_PAD = NTILES * ROWS_PER_TILE * 128
NOUT = HI * WI * CI
NOUT_PAD = 788608
OUT_CHUNK = NOUT_PAD // NTILES
STAGE_SZ = 8192
STAGE_TAIL = OUT_CHUNK - 6 * STAGE_SZ


def _tc_argmax_body(x_ref, w_ref, out_ref, slab_ref):
    ho = pl.program_id(0)
    for dh in range(FSIZE):
        row = x_ref[2 * ho + dh]
        pairs = row[0:110].reshape(WO, 2, CI)
        e = pairs[:, 0, :]
        o = pairs[:, 1, :]
        e2 = jnp.concatenate([e[1:], row[110:111]], axis=0)
        slab_ref[dh * FSIZE + 0] = e
        slab_ref[dh * FSIZE + 1] = o
        slab_ref[dh * FSIZE + 2] = e2

    def body(k, carry):
        maxv, maxp = carry
        sk = slab_ref[k]
        wk = w_ref[k]
        pbase = k * CI
        for ci in range(CI):
            prod = sk[:, ci:ci + 1] * wk[ci:ci + 1, :]
            upd = prod > maxv
            maxv = jnp.where(upd, prod, maxv)
            maxp = jnp.where(upd, pbase + ci, maxp)
        return maxv, maxp

    neg_inf = jnp.float32(-jnp.inf)
    maxv0 = jnp.full((WO, CO), neg_inf, dtype=jnp.float32)
    maxp0 = jnp.zeros((WO, CO), dtype=jnp.int32)
    _, maxp = lax.fori_loop(0, FSIZE * FSIZE, body, (maxv0, maxp0))
    ph = maxp // (FSIZE * CI)
    pw = (maxp // CI) % FSIZE
    pc = maxp % CI
    wo = lax.broadcasted_iota(jnp.int32, (WO, CO), 0)
    gh = ph + 2 * ho
    gw = pw + 2 * wo
    out_ref[0] = gh * ROWW + gw * CI + pc


def _tc_argmax(x, w):
    return pl.pallas_call(
        _tc_argmax_body,
        grid=(HO,),
        in_specs=[
            pl.BlockSpec((HI, WI, CI), lambda i: (0, 0, 0)),
            pl.BlockSpec((FSIZE * FSIZE, CI, CO), lambda i: (0, 0, 0)),
        ],
        out_specs=pl.BlockSpec((1, WO, CO), lambda i: (i, 0, 0)),
        out_shape=jax.ShapeDtypeStruct((HO, WO, CO), jnp.int32),
        scratch_shapes=[pltpu.VMEM((FSIZE * FSIZE, WO, CI), jnp.float32)],
    )(x, w)


def _sc_scatter_body(idx_hbm, val_hbm, out_hbm, idx_v, val_v, stage, acc):
    c = lax.axis_index("c")
    s = lax.axis_index("s")

    @pl.when(c == 0)
    def _():
        zeros16 = jnp.zeros((16,), jnp.float32)

        def zbody(j, carry):
            stage[pl.ds(j * 16, 16)] = zeros16
            return carry

        lax.fori_loop(0, STAGE_SZ // 16, zbody, jnp.int32(0))
        base = s * OUT_CHUNK
        for q in range(6):
            pltpu.sync_copy(stage, acc.at[pl.ds(base + q * STAGE_SZ,
                                                STAGE_SZ)])
        pltpu.sync_copy(stage.at[pl.ds(0, STAGE_TAIL)],
                        acc.at[pl.ds(base + 6 * STAGE_SZ, STAGE_TAIL)])
        pltpu.sync_copy(idx_hbm.at[pl.ds(s * ROWS_PER_TILE, ROWS_PER_TILE)],
                        idx_v)
        pltpu.sync_copy(val_hbm.at[pl.ds(s * ROWS_PER_TILE, ROWS_PER_TILE)],
                        val_v)
        plsc.subcore_barrier()

        def body(j, carry):
            pltpu.sync_copy(val_v.at[j], acc.at[idx_v.at[j]], add=True)
            return carry

        lax.fori_loop(0, ROWS_PER_TILE, body, jnp.int32(0))
        plsc.subcore_barrier()
        for q in range(6):
            pltpu.sync_copy(acc.at[pl.ds(base + q * STAGE_SZ, STAGE_SZ)],
                            stage)
            pltpu.sync_copy(stage,
                            out_hbm.at[pl.ds(base + q * STAGE_SZ, STAGE_SZ)])
        pltpu.sync_copy(acc.at[pl.ds(base + 6 * STAGE_SZ, STAGE_TAIL)],
                        stage.at[pl.ds(0, STAGE_TAIL)])
        pltpu.sync_copy(stage.at[pl.ds(0, STAGE_TAIL)],
                        out_hbm.at[pl.ds(base + 6 * STAGE_SZ, STAGE_TAIL)])


@functools.cache
def _sc_scatter_kernel():
    return pl.kernel(
        _sc_scatter_body,
        mesh=plsc.VectorSubcoreMesh(
            core_axis_name="c", subcore_axis_name="s", num_cores=2),
        out_type=jax.ShapeDtypeStruct((NOUT_PAD,), jnp.float32),
        scratch_types=[
            pltpu.VMEM((ROWS_PER_TILE, 128), jnp.int32),
            pltpu.VMEM((ROWS_PER_TILE, 128), jnp.float32),
            pltpu.VMEM((STAGE_SZ,), jnp.float32),
            pltpu.VMEM_SHARED((NOUT_PAD,), jnp.float32),
        ],
    )


def kernel(inputs, layer_output, layer_weights):
    x = layer_output[0]
    w = layer_weights.reshape(FSIZE * FSIZE, CI, CO)
    gidx = _tc_argmax(x, w)
    idx_flat = gidx.reshape(-1)
    val_flat = inputs.reshape(-1)
    pad = PAIRS_PAD - PAIRS
    idx2d = jnp.concatenate(
        [idx_flat, jnp.zeros((pad,), jnp.int32)]).reshape(-1, 128)
    val2d = jnp.concatenate(
        [val_flat, jnp.zeros((pad,), jnp.float32)]).reshape(-1, 128)
    out = _sc_scatter_kernel()(idx2d, val2d)
    return out[:NOUT].reshape(1, HI, WI, CI)


# speedup vs baseline: 4.2559x; 1.0155x over previous
import functools

import jax
import jax.numpy as jnp
from jax import lax
from jax.experimental import pallas as pl
from jax.experimental.pallas import tpu as pltpu
from jax.experimental.pallas import tpu_sc as plsc

STRIDE = 2
FSIZE = 3
HO = WO = 55
HI = WI = 111
CI = 64
CO = 128
P = FSIZE * FSIZE * CI
ROWW = WI * CI

NTILES = 16
PAIRS = HO * WO * CO
ROWS_PER_TILE = 192
PAIRS_PAD = NTILES * ROWS_PER_TILE * 128
NOUT = HI * WI * CI
NOUT_PAD = 788608
OUT_CHUNK = NOUT_PAD // NTILES
STAGE_SZ = 8192
STAGE_TAIL = OUT_CHUNK - 6 * STAGE_SZ


WOP = 56


def _tc_argmax_body(x_ref, w_ref, out_ref, slab_ref):
    ho = pl.program_id(0)
    for dh in range(FSIZE):
        row = x_ref[2 * ho + dh]
        pairs = row.reshape(WOP, 2, CI)
        e = pairs[:, 0, :]
        o = pairs[:, 1, :]
        e2 = jnp.concatenate([e[1:], e[:1]], axis=0)
        slab_ref[dh * FSIZE + 0] = e
        slab_ref[dh * FSIZE + 1] = o
        slab_ref[dh * FSIZE + 2] = e2

    NCH = 2

    def body(k, carry):
        mv, mp = carry
        sk = slab_ref[k]
        wk = w_ref[k]
        pbase = k * CI
        mv, mp = list(mv), list(mp)
        for a in range(NCH):
            for ci in range(a * (CI // NCH), (a + 1) * (CI // NCH)):
                prod = sk[:, ci:ci + 1] * wk[ci:ci + 1, :]
                upd = prod > mv[a]
                mv[a] = jnp.maximum(mv[a], prod)
                mp[a] = jnp.where(upd, pbase + ci, mp[a])
        return tuple(mv), tuple(mp)

    neg_inf = jnp.float32(-jnp.inf)
    mv0 = tuple(jnp.full((WOP, CO), neg_inf, dtype=jnp.float32)
                for _ in range(NCH))
    mp0 = tuple(jnp.zeros((WOP, CO), dtype=jnp.int32) for _ in range(NCH))
    mv, mp = lax.fori_loop(0, FSIZE * FSIZE, body, (mv0, mp0))
    maxv, maxp = mv[0], mp[0]
    for a in range(1, NCH):
        upd = (mv[a] > maxv) | ((mv[a] == maxv) & (mp[a] < maxp))
        maxv = jnp.where(upd, mv[a], maxv)
        maxp = jnp.where(upd, mp[a], maxp)
    ph = maxp // (FSIZE * CI)
    pw = (maxp // CI) % FSIZE
    pc = maxp % CI
    wo = lax.broadcasted_iota(jnp.int32, (WOP, CO), 0)
    gh = ph + 2 * ho
    gw = pw + 2 * wo
    gidx = gh * ROWW + gw * CI + pc
    out_ref[0] = gidx[:WO]


def _tc_argmax(x, w):
    return pl.pallas_call(
        _tc_argmax_body,
        grid=(HO,),
        in_specs=[
            pl.BlockSpec((HI, WI + 1, CI), lambda i: (0, 0, 0)),
            pl.BlockSpec((FSIZE * FSIZE, CI, CO), lambda i: (0, 0, 0)),
        ],
        out_specs=pl.BlockSpec((1, WO, CO), lambda i: (i, 0, 0)),
        out_shape=jax.ShapeDtypeStruct((HO, WO, CO), jnp.int32),
        scratch_shapes=[pltpu.VMEM((FSIZE * FSIZE, WOP, CI), jnp.float32)],
    )(x, w)


def _sc_scatter_body(idx_hbm, val_hbm, out_hbm, idx_v, val_v, stage, acc):
    c = lax.axis_index("c")
    s = lax.axis_index("s")

    @pl.when(c == 0)
    def _():
        zeros16 = jnp.zeros((16,), jnp.float32)

        def zbody(j, carry):
            stage[pl.ds(j * 16, 16)] = zeros16
            return carry

        lax.fori_loop(0, STAGE_SZ // 16, zbody, jnp.int32(0))
        base = s * OUT_CHUNK
        for q in range(6):
            pltpu.sync_copy(stage, acc.at[pl.ds(base + q * STAGE_SZ,
                                                STAGE_SZ)])
        pltpu.sync_copy(stage.at[pl.ds(0, STAGE_TAIL)],
                        acc.at[pl.ds(base + 6 * STAGE_SZ, STAGE_TAIL)])
        pltpu.sync_copy(idx_hbm.at[pl.ds(s * ROWS_PER_TILE, ROWS_PER_TILE)],
                        idx_v)
        pltpu.sync_copy(val_hbm.at[pl.ds(s * ROWS_PER_TILE, ROWS_PER_TILE)],
                        val_v)
        plsc.subcore_barrier()

        def body(j, carry):
            pltpu.sync_copy(val_v.at[j], acc.at[idx_v.at[j]], add=True)
            return carry

        lax.fori_loop(0, ROWS_PER_TILE, body, jnp.int32(0))
        plsc.subcore_barrier()
        for q in range(6):
            pltpu.sync_copy(acc.at[pl.ds(base + q * STAGE_SZ, STAGE_SZ)],
                            stage)
            pltpu.sync_copy(stage,
                            out_hbm.at[pl.ds(base + q * STAGE_SZ, STAGE_SZ)])
        pltpu.sync_copy(acc.at[pl.ds(base + 6 * STAGE_SZ, STAGE_TAIL)],
                        stage.at[pl.ds(0, STAGE_TAIL)])
        pltpu.sync_copy(stage.at[pl.ds(0, STAGE_TAIL)],
                        out_hbm.at[pl.ds(base + 6 * STAGE_SZ, STAGE_TAIL)])


@functools.cache
def _sc_scatter_kernel():
    return pl.kernel(
        _sc_scatter_body,
        mesh=plsc.VectorSubcoreMesh(
            core_axis_name="c", subcore_axis_name="s", num_cores=2),
        out_type=jax.ShapeDtypeStruct((NOUT_PAD,), jnp.float32),
        scratch_types=[
            pltpu.VMEM((ROWS_PER_TILE, 128), jnp.int32),
            pltpu.VMEM((ROWS_PER_TILE, 128), jnp.float32),
            pltpu.VMEM((STAGE_SZ,), jnp.float32),
            pltpu.VMEM_SHARED((NOUT_PAD,), jnp.float32),
        ],
    )


def kernel(inputs, layer_output, layer_weights):
    x = jnp.pad(layer_output[0], ((0, 0), (0, 1), (0, 0)))
    w = layer_weights.reshape(FSIZE * FSIZE, CI, CO)
    gidx = _tc_argmax(x, w)
    idx_flat = gidx.reshape(-1)
    val_flat = inputs.reshape(-1)
    pad = PAIRS_PAD - PAIRS
    idx2d = jnp.concatenate(
        [idx_flat, jnp.zeros((pad,), jnp.int32)]).reshape(-1, 128)
    val2d = jnp.concatenate(
        [val_flat, jnp.zeros((pad,), jnp.float32)]).reshape(-1, 128)
    out = _sc_scatter_kernel()(idx2d, val2d)
    return out[:NOUT].reshape(1, HI, WI, CI)


# speedup vs baseline: 4.2801x; 1.0057x over previous
import functools

import jax
import jax.numpy as jnp
from jax import lax
from jax.experimental import pallas as pl
from jax.experimental.pallas import tpu as pltpu
from jax.experimental.pallas import tpu_sc as plsc

STRIDE = 2
FSIZE = 3
HO = WO = 55
HI = WI = 111
CI = 64
CO = 128
P = FSIZE * FSIZE * CI
ROWW = WI * CI

NG = 28
WOP = 56
LAN = 2 * WOP

NTILES = 16
PAIRS = NG * CO * LAN
ROWS_PER_TILE = 200
PAIRS_PAD = NTILES * ROWS_PER_TILE * 128
NOUT = HI * WI * CI
NOUT_PAD = 788608
OUT_CHUNK = NOUT_PAD // NTILES
STAGE_SZ = 8192
STAGE_TAIL = OUT_CHUNK - 6 * STAGE_SZ


def _tc_argmax_body(xp_ref, wb_ref, out_ref):
    g = pl.program_id(0)

    def body(k, carry):
        mv, mp = carry
        pbase = k * CI
        for ci in range(CI):
            srow = xp_ref[0, k, ci]
            sb = jnp.broadcast_to(srow[None, :], (CO, LAN))
            prod = wb_ref[pbase + ci] * sb
            upd = prod > mv
            mv = jnp.maximum(mv, prod)
            mp = jnp.where(upd, pbase + ci, mp)
        return mv, mp

    neg_inf = jnp.float32(-jnp.inf)
    mv0 = jnp.full((CO, LAN), neg_inf, dtype=jnp.float32)
    mp0 = jnp.zeros((CO, LAN), dtype=jnp.int32)
    _, maxp = lax.fori_loop(0, FSIZE * FSIZE, body, (mv0, mp0))
    ph = maxp // (FSIZE * CI)
    pw = (maxp // CI) % FSIZE
    pc = maxp % CI
    lane = lax.broadcasted_iota(jnp.int32, (CO, LAN), 1)
    hoi = lane // WOP
    wo = lane % WOP
    gh = ph + 2 * (2 * g + hoi)
    gw = pw + 2 * wo
    gidx = gh * ROWW + gw * CI + pc
    out_ref[0] = jnp.minimum(gidx, NOUT - 1)


def _tc_argmax(xp, wb):
    return pl.pallas_call(
        _tc_argmax_body,
        grid=(NG,),
        in_specs=[
            pl.BlockSpec((1, FSIZE * FSIZE, CI, LAN), lambda i: (i, 0, 0, 0)),
            pl.BlockSpec((P, CO, LAN), lambda i: (0, 0, 0)),
        ],
        out_specs=pl.BlockSpec((1, CO, LAN), lambda i: (i, 0, 0)),
        out_shape=jax.ShapeDtypeStruct((NG, CO, LAN), jnp.int32),
    )(xp, wb)


def _sc_scatter_body(idx_hbm, val_hbm, out_hbm, idx_v, val_v, stage, acc):
    c = lax.axis_index("c")
    s = lax.axis_index("s")

    @pl.when(c == 0)
    def _():
        zeros16 = jnp.zeros((16,), jnp.float32)

        def zbody(j, carry):
            stage[pl.ds(j * 16, 16)] = zeros16
            return carry

        lax.fori_loop(0, STAGE_SZ // 16, zbody, jnp.int32(0))
        base = s * OUT_CHUNK
        for q in range(6):
            pltpu.sync_copy(stage, acc.at[pl.ds(base + q * STAGE_SZ,
                                                STAGE_SZ)])
        pltpu.sync_copy(stage.at[pl.ds(0, STAGE_TAIL)],
                        acc.at[pl.ds(base + 6 * STAGE_SZ, STAGE_TAIL)])
        pltpu.sync_copy(idx_hbm.at[pl.ds(s * ROWS_PER_TILE, ROWS_PER_TILE)],
                        idx_v)
        pltpu.sync_copy(val_hbm.at[pl.ds(s * ROWS_PER_TILE, ROWS_PER_TILE)],
                        val_v)
        plsc.subcore_barrier()

        def body(j, carry):
            pltpu.sync_copy(val_v.at[j], acc.at[idx_v.at[j]], add=True)
            return carry

        lax.fori_loop(0, ROWS_PER_TILE, body, jnp.int32(0))
        plsc.subcore_barrier()
        for q in range(6):
            pltpu.sync_copy(acc.at[pl.ds(base + q * STAGE_SZ, STAGE_SZ)],
                            stage)
            pltpu.sync_copy(stage,
                            out_hbm.at[pl.ds(base + q * STAGE_SZ, STAGE_SZ)])
        pltpu.sync_copy(acc.at[pl.ds(base + 6 * STAGE_SZ, STAGE_TAIL)],
                        stage.at[pl.ds(0, STAGE_TAIL)])
        pltpu.sync_copy(stage.at[pl.ds(0, STAGE_TAIL)],
                        out_hbm.at[pl.ds(base + 6 * STAGE_SZ, STAGE_TAIL)])


@functools.cache
def _sc_scatter_kernel():
    return pl.kernel(
        _sc_scatter_body,
        mesh=plsc.VectorSubcoreMesh(
            core_axis_name="c", subcore_axis_name="s", num_cores=2),
        out_type=jax.ShapeDtypeStruct((NOUT_PAD,), jnp.float32),
        scratch_types=[
            pltpu.VMEM((ROWS_PER_TILE, 128), jnp.int32),
            pltpu.VMEM((ROWS_PER_TILE, 128), jnp.float32),
            pltpu.VMEM((STAGE_SZ,), jnp.float32),
            pltpu.VMEM_SHARED((NOUT_PAD,), jnp.float32),
        ],
    )


def _prep_xp(x):
    xd = jnp.pad(x.transpose(0, 2, 1), ((0, 5), (0, 0), (0, 3)))
    variants = [xd[:, :, dw:dw + 112:2] for dw in range(FSIZE)]
    parts = []
    for dh in range(FSIZE):
        for dw in range(FSIZE):
            v = variants[dw]
            a = v[dh:dh + 112].reshape(NG, 4, CI, WOP)[:, 0]
            b = v[dh + 2:dh + 114].reshape(NG, 4, CI, WOP)[:, 0]
            parts.append(jnp.concatenate([a, b], axis=-1))
    return jnp.stack(parts, axis=1)


def kernel(inputs, layer_output, layer_weights):
    x = layer_output[0]
    xp = _prep_xp(x)
    w2 = layer_weights.reshape(P, CO)
    wb = jnp.broadcast_to(w2[:, :, None], (P, CO, LAN))
    gidx = _tc_argmax(xp, wb)
    t = jnp.pad(inputs[0], ((0, 1), (0, 1), (0, 0)))
    vals = t.reshape(NG, 2, WOP, CO).transpose(0, 3, 1, 2).reshape(-1)
    idx_flat = gidx.reshape(-1)
    pad = PAIRS_PAD - PAIRS
    idx2d = jnp.concatenate(
        [idx_flat, jnp.zeros((pad,), jnp.int32)]).reshape(-1, 128)
    val2d = jnp.concatenate(
        [vals, jnp.zeros((pad,), jnp.float32)]).reshape(-1, 128)
    out = _sc_scatter_kernel()(idx2d, val2d)
    return out[:NOUT].reshape(1, HI, WI, CI)


# speedup vs baseline: 4.4273x; 1.0344x over previous
import functools

import jax
import jax.numpy as jnp
from jax import lax
from jax.experimental import pallas as pl
from jax.experimental.pallas import tpu as pltpu
from jax.experimental.pallas import tpu_sc as plsc

STRIDE = 2
FSIZE = 3
HO = WO = 55
HI = WI = 111
CI = 64
CO = 128
P = FSIZE * FSIZE * CI
ROWW = WI * CI

NG = 28
WOP = 56
LAN = 2 * WOP

NTILES = 16
PAIRS = NG * CO * LAN
ROWS_PER_TILE = 200
PAIRS_PAD = NTILES * ROWS_PER_TILE * 128
NOUT = HI * WI * CI
NOUT_PAD = 788608
OUT_CHUNK = NOUT_PAD // NTILES
STAGE_SZ = 8192
STAGE_TAIL = OUT_CHUNK - 6 * STAGE_SZ


def _tc_argmax_body(xp_ref, wt_ref, out_ref, wb_ref):
    g = pl.program_id(0)

    @pl.when(g == 0)
    def _build_wb():
        for p in range(P):
            col = wt_ref[:, p:p + 1]
            wb_ref[p] = jnp.broadcast_to(col, (CO, LAN))

    def body(k, carry):
        mv, mp = carry
        pbase = k * CI
        for ci in range(CI):
            srow = xp_ref[0, k, ci]
            sb = jnp.broadcast_to(srow[None, :], (CO, LAN))
            prod = wb_ref[pbase + ci] * sb
            upd = prod > mv
            mv = jnp.maximum(mv, prod)
            mp = jnp.where(upd, pbase + ci, mp)
        return mv, mp

    neg_inf = jnp.float32(-jnp.inf)
    mv0 = jnp.full((CO, LAN), neg_inf, dtype=jnp.float32)
    mp0 = jnp.zeros((CO, LAN), dtype=jnp.int32)
    _, maxp = lax.fori_loop(0, FSIZE * FSIZE, body, (mv0, mp0))
    ph = maxp // (FSIZE * CI)
    pw = (maxp // CI) % FSIZE
    pc = maxp % CI
    lane = lax.broadcasted_iota(jnp.int32, (CO, LAN), 1)
    hoi = lane // WOP
    wo = lane % WOP
    gh = ph + 2 * (2 * g + hoi)
    gw = pw + 2 * wo
    gidx = gh * ROWW + gw * CI + pc
    out_ref[0] = jnp.minimum(gidx, NOUT - 1)


def _tc_argmax(xp, wt):
    return pl.pallas_call(
        _tc_argmax_body,
        grid=(NG,),
        in_specs=[
            pl.BlockSpec((1, FSIZE * FSIZE, CI, LAN), lambda i: (i, 0, 0, 0)),
            pl.BlockSpec((CO, P), lambda i: (0, 0)),
        ],
        out_specs=pl.BlockSpec((1, CO, LAN), lambda i: (i, 0, 0)),
        out_shape=jax.ShapeDtypeStruct((NG, CO, LAN), jnp.int32),
        scratch_shapes=[pltpu.VMEM((P, CO, LAN), jnp.float32)],
    )(xp, wt)


def _sc_scatter_body(idx_hbm, val_hbm, out_hbm, idx_v, val_v, stage, acc):
    c = lax.axis_index("c")
    s = lax.axis_index("s")

    @pl.when(c == 0)
    def _():
        zeros16 = jnp.zeros((16,), jnp.float32)

        def zbody(j, carry):
            stage[pl.ds(j * 16, 16)] = zeros16
            return carry

        lax.fori_loop(0, STAGE_SZ // 16, zbody, jnp.int32(0))
        base = s * OUT_CHUNK
        for q in range(6):
            pltpu.sync_copy(stage, acc.at[pl.ds(base + q * STAGE_SZ,
                                                STAGE_SZ)])
        pltpu.sync_copy(stage.at[pl.ds(0, STAGE_TAIL)],
                        acc.at[pl.ds(base + 6 * STAGE_SZ, STAGE_TAIL)])
        pltpu.sync_copy(idx_hbm.at[pl.ds(s * ROWS_PER_TILE, ROWS_PER_TILE)],
                        idx_v)
        pltpu.sync_copy(val_hbm.at[pl.ds(s * ROWS_PER_TILE, ROWS_PER_TILE)],
                        val_v)
        plsc.subcore_barrier()

        def body(j, carry):
            pltpu.sync_copy(val_v.at[j], acc.at[idx_v.at[j]], add=True)
            return carry

        lax.fori_loop(0, ROWS_PER_TILE, body, jnp.int32(0))
        plsc.subcore_barrier()
        for q in range(6):
            pltpu.sync_copy(acc.at[pl.ds(base + q * STAGE_SZ, STAGE_SZ)],
                            stage)
            pltpu.sync_copy(stage,
                            out_hbm.at[pl.ds(base + q * STAGE_SZ, STAGE_SZ)])
        pltpu.sync_copy(acc.at[pl.ds(base + 6 * STAGE_SZ, STAGE_TAIL)],
                        stage.at[pl.ds(0, STAGE_TAIL)])
        pltpu.sync_copy(stage.at[pl.ds(0, STAGE_TAIL)],
                        out_hbm.at[pl.ds(base + 6 * STAGE_SZ, STAGE_TAIL)])


@functools.cache
def _sc_scatter_kernel():
    return pl.kernel(
        _sc_scatter_body,
        mesh=plsc.VectorSubcoreMesh(
            core_axis_name="c", subcore_axis_name="s", num_cores=2),
        out_type=jax.ShapeDtypeStruct((NOUT_PAD,), jnp.float32),
        scratch_types=[
            pltpu.VMEM((ROWS_PER_TILE, 128), jnp.int32),
            pltpu.VMEM((ROWS_PER_TILE, 128), jnp.float32),
            pltpu.VMEM((STAGE_SZ,), jnp.float32),
            pltpu.VMEM_SHARED((NOUT_PAD,), jnp.float32),
        ],
    )


def _prep_xp(x):
    xd = jnp.pad(x.transpose(0, 2, 1), ((0, 5), (0, 0), (0, 3)))
    variants = [xd[:, :, dw:dw + 112:2] for dw in range(FSIZE)]
    parts = []
    for dh in range(FSIZE):
        for dw in range(FSIZE):
            v = variants[dw]
            a = v[dh:dh + 112].reshape(NG, 4, CI, WOP)[:, 0]
            b = v[dh + 2:dh + 114].reshape(NG, 4, CI, WOP)[:, 0]
            parts.append(jnp.concatenate([a, b], axis=-1))
    return jnp.stack(parts, axis=1)


def kernel(inputs, layer_output, layer_weights):
    x = layer_output[0]
    xp = _prep_xp(x)
    wt = layer_weights.reshape(P, CO).T
    gidx = _tc_argmax(xp, wt)
    t = jnp.pad(inputs[0], ((0, 1), (0, 1), (0, 0)))
    vals = t.reshape(NG, 2, WOP, CO).transpose(0, 3, 1, 2).reshape(-1)
    idx_flat = gidx.reshape(-1)
    pad = PAIRS_PAD - PAIRS
    idx2d = jnp.concatenate(
        [idx_flat, jnp.zeros((pad,), jnp.int32)]).reshape(-1, 128)
    val2d = jnp.concatenate(
        [vals, jnp.zeros((pad,), jnp.float32)]).reshape(-1, 128)
    out = _sc_scatter_kernel()(idx2d, val2d)
    return out[:NOUT].reshape(1, HI, WI, CI)


# speedup vs baseline: 6.1023x; 1.3783x over previous
import functools

import jax
import jax.numpy as jnp
from jax import lax
from jax.experimental import pallas as pl
from jax.experimental.pallas import tpu as pltpu
from jax.experimental.pallas import tpu_sc as plsc

STRIDE = 2
FSIZE = 3
HO = WO = 55
HI = WI = 111
CI = 64
CO = 128
P = FSIZE * FSIZE * CI
ROWW = WI * CI

NG = 28
WOP = 56
LAN = 2 * WOP

NTILES = 16
SC_TILES = 14
PAIRS = NG * CO * LAN
ROWS_PER_TILE = 224
NOUT = HI * WI * CI
NOUT_PAD = 788608
OUT_CHUNK = NOUT_PAD // NTILES
STAGE_SZ = 8192
STAGE_TAIL = OUT_CHUNK - 6 * STAGE_SZ


def _tc_argmax_body(xs_ref, wt_ref, out_ref, wb_ref, slab_ref):
    g = pl.program_id(0)

    @pl.when(g == 0)
    def _build_wb():
        for p in range(P):
            col = wt_ref[:, p:p + 1]
            wb_ref[p] = jnp.broadcast_to(col, (CO, LAN))

    for dh in range(FSIZE):
        for dw in range(FSIZE):
            par, off = (dw % 2, dw // 2)
            a = xs_ref[4 * g + dh, :, par, pl.ds(off, WOP)]
            b = xs_ref[4 * g + 2 + dh, :, par, pl.ds(off, WOP)]
            slab_ref[dh * FSIZE + dw] = jnp.concatenate([a, b], axis=1)

    def body(k, carry):
        mv, mp = carry
        pbase = k * CI
        for ci in range(CI):
            srow = slab_ref[k, ci]
            sb = jnp.broadcast_to(srow[None, :], (CO, LAN))
            prod = wb_ref[pbase + ci] * sb
            upd = prod > mv
            mv = jnp.maximum(mv, prod)
            mp = jnp.where(upd, pbase + ci, mp)
        return mv, mp

    neg_inf = jnp.float32(-jnp.inf)
    mv0 = jnp.full((CO, LAN), neg_inf, dtype=jnp.float32)
    mp0 = jnp.zeros((CO, LAN), dtype=jnp.int32)
    _, maxp = lax.fori_loop(0, FSIZE * FSIZE, body, (mv0, mp0))
    ph = maxp // (FSIZE * CI)
    pw = (maxp // CI) % FSIZE
    pc = maxp % CI
    lane = lax.broadcasted_iota(jnp.int32, (CO, LAN), 1)
    hoi = lane // WOP
    wo = lane % WOP
    gh = ph + 2 * (2 * g + hoi)
    gw = pw + 2 * wo
    gidx = gh * ROWW + gw * CI + pc
    out_ref[0] = jnp.minimum(gidx, NOUT - 1)


def _tc_argmax(xs, wt):
    return pl.pallas_call(
        _tc_argmax_body,
        grid=(NG,),
        in_specs=[
            pl.BlockSpec((116, CI, 2, 57), lambda i: (0, 0, 0, 0)),
            pl.BlockSpec((CO, P), lambda i: (0, 0)),
        ],
        out_specs=pl.BlockSpec((1, CO, LAN), lambda i: (i, 0, 0)),
        out_shape=jax.ShapeDtypeStruct((NG, CO, LAN), jnp.int32),
        scratch_shapes=[
            pltpu.VMEM((P, CO, LAN), jnp.float32),
            pltpu.VMEM((FSIZE * FSIZE, CI, LAN), jnp.float32),
        ],
    )(xs, wt)


def _sc_scatter_body(idx_hbm, val_hbm, out_hbm, idx_v, val_v, stage, acc):
    c = lax.axis_index("c")
    s = lax.axis_index("s")

    @pl.when(c == 0)
    def _():
        zeros16 = jnp.zeros((16,), jnp.float32)

        def zbody(j, carry):
            stage[pl.ds(j * 16, 16)] = zeros16
            return carry

        lax.fori_loop(0, STAGE_SZ // 16, zbody, jnp.int32(0))
        base = s * OUT_CHUNK
        for q in range(6):
            pltpu.sync_copy(stage, acc.at[pl.ds(base + q * STAGE_SZ,
                                                STAGE_SZ)])
        pltpu.sync_copy(stage.at[pl.ds(0, STAGE_TAIL)],
                        acc.at[pl.ds(base + 6 * STAGE_SZ, STAGE_TAIL)])

        @pl.when(s < SC_TILES)
        def _load():
            pltpu.sync_copy(
                idx_hbm.at[pl.ds(s * ROWS_PER_TILE, ROWS_PER_TILE)], idx_v)
            pltpu.sync_copy(
                val_hbm.at[pl.ds(s * ROWS_PER_TILE, ROWS_PER_TILE)], val_v)

        plsc.subcore_barrier()

        @pl.when(s < SC_TILES)
        def _scatter():
            def body(j, carry):
                pltpu.sync_copy(val_v.at[j], acc.at[idx_v.at[j]], add=True)
                return carry

            lax.fori_loop(0, ROWS_PER_TILE, body, jnp.int32(0))

        plsc.subcore_barrier()
        for q in range(6):
            pltpu.sync_copy(acc.at[pl.ds(base + q * STAGE_SZ, STAGE_SZ)],
                            stage)
            pltpu.sync_copy(stage,
                            out_hbm.at[pl.ds(base + q * STAGE_SZ, STAGE_SZ)])
        pltpu.sync_copy(acc.at[pl.ds(base + 6 * STAGE_SZ, STAGE_TAIL)],
                        stage.at[pl.ds(0, STAGE_TAIL)])
        pltpu.sync_copy(stage.at[pl.ds(0, STAGE_TAIL)],
                        out_hbm.at[pl.ds(base + 6 * STAGE_SZ, STAGE_TAIL)])


@functools.cache
def _sc_scatter_kernel():
    return pl.kernel(
        _sc_scatter_body,
        mesh=plsc.VectorSubcoreMesh(
            core_axis_name="c", subcore_axis_name="s", num_cores=2),
        out_type=jax.ShapeDtypeStruct((NOUT_PAD,), jnp.float32),
        scratch_types=[
            pltpu.VMEM((ROWS_PER_TILE, 128), jnp.int32),
            pltpu.VMEM((ROWS_PER_TILE, 128), jnp.float32),
            pltpu.VMEM((STAGE_SZ,), jnp.float32),
            pltpu.VMEM_SHARED((NOUT_PAD,), jnp.float32),
        ],
    )


def kernel(inputs, layer_output, layer_weights):
    x = layer_output[0]
    xs = jnp.pad(x, ((0, 5), (0, 3), (0, 0))).reshape(116, 57, 2, CI)
    xs = xs.transpose(0, 3, 2, 1)
    wt = layer_weights.reshape(P, CO).T
    gidx = _tc_argmax(xs, wt)
    t = jnp.pad(inputs[0], ((0, 1), (0, 1), (0, 0)))
    vals = t.reshape(NG, 2, WOP, CO).transpose(0, 3, 1, 2)
    idx2d = gidx.reshape(-1, 128)
    val2d = vals.reshape(-1, 128)
    out = _sc_scatter_kernel()(idx2d, val2d)
    return out[:NOUT].reshape(1, HI, WI, CI)


# speedup vs baseline: 6.4467x; 1.0564x over previous
import functools

import jax
import jax.numpy as jnp
from jax import lax
from jax.experimental import pallas as pl
from jax.experimental.pallas import tpu as pltpu
from jax.experimental.pallas import tpu_sc as plsc

STRIDE = 2
FSIZE = 3
HO = WO = 55
HI = WI = 111
CI = 64
CO = 128
P = FSIZE * FSIZE * CI
ROWW = WI * CI

NG = 28
WOP = 56
LAN = 2 * WOP

NTILES = 16
SC_TILES = 14
PAIRS = NG * CO * LAN
ROWS_PER_TILE = 224
NOUT = HI * WI * CI
NOUT_PAD = 788608
OUT_CHUNK = NOUT_PAD // NTILES
STAGE_SZ = 8192
STAGE_TAIL = OUT_CHUNK - 6 * STAGE_SZ


def _tc_argmax_body(xs_ref, wt_ref, out_ref, wb_ref, slab_ref):
    g = pl.program_id(0)

    @pl.when(g == 0)
    def _build_wb():
        for p in range(P):
            col = wt_ref[:, p:p + 1]
            wb_ref[p] = jnp.broadcast_to(col, (CO, LAN))

    for dh in range(FSIZE):
        for dw in range(FSIZE):
            par, off = (dw % 2, dw // 2)
            a = xs_ref[4 * g + dh, :, par, pl.ds(off, WOP)]
            b = xs_ref[4 * g + 2 + dh, :, par, pl.ds(off, WOP)]
            slab_ref[dh * FSIZE + dw] = jnp.concatenate([a, b], axis=1)

    def body(k, carry):
        mv, mp = carry
        pbase = k * CI
        for ci in range(CI):
            srow = slab_ref[k, ci]
            sb = jnp.broadcast_to(srow[None, :], (CO, LAN))
            prod = wb_ref[pbase + ci] * sb
            upd = prod > mv
            mv = jnp.maximum(mv, prod)
            mp = jnp.where(upd, pbase + ci, mp)
        return mv, mp

    neg_inf = jnp.float32(-jnp.inf)
    mv0 = jnp.full((CO, LAN), neg_inf, dtype=jnp.float32)
    mp0 = jnp.zeros((CO, LAN), dtype=jnp.int32)
    _, maxp = lax.fori_loop(0, FSIZE * FSIZE, body, (mv0, mp0))
    ph = maxp // (FSIZE * CI)
    pw = (maxp // CI) % FSIZE
    pc = maxp % CI
    lane = lax.broadcasted_iota(jnp.int32, (CO, LAN), 1)
    hoi = lane // WOP
    wo = lane % WOP
    gh = ph + 2 * (2 * g + hoi)
    gw = pw + 2 * wo
    gidx = gh * ROWW + gw * CI + pc
    out_ref[0] = jnp.minimum(gidx, NOUT - 1)


def _tc_argmax(xs, wt):
    return pl.pallas_call(
        _tc_argmax_body,
        grid=(NG,),
        in_specs=[
            pl.BlockSpec((116, CI, 2, 57), lambda i: (0, 0, 0, 0)),
            pl.BlockSpec((CO, P), lambda i: (0, 0)),
        ],
        out_specs=pl.BlockSpec((1, CO, LAN), lambda i: (i, 0, 0)),
        out_shape=jax.ShapeDtypeStruct((NG, CO, LAN), jnp.int32),
        scratch_shapes=[
            pltpu.VMEM((P, CO, LAN), jnp.float32),
            pltpu.VMEM((FSIZE * FSIZE, CI, LAN), jnp.float32),
        ],
    )(xs, wt)


WAVE = 8
DRAIN_CH = STAGE_SZ // 2
N_DRAIN = 12


def _sc_scatter_body(idx_hbm, val_hbm, out_hbm, idx_v, val_v, stage,
                     acc, sem_z, sem_l, sem_s, sem_i0, sem_i1, sem_o0,
                     sem_o1):
    c = lax.axis_index("c")
    s = lax.axis_index("s")

    @pl.when(c == 0)
    def _():
        zeros16 = jnp.zeros((16,), jnp.float32)

        def zbody(j, carry):
            stage[pl.ds(j * 16, 16)] = zeros16
            return carry

        lax.fori_loop(0, STAGE_SZ // 16, zbody, jnp.int32(0))
        base = s * OUT_CHUNK
        zd = [pltpu.async_copy(
            stage, acc.at[pl.ds(base + q * STAGE_SZ, STAGE_SZ)], sem_z)
            for q in range(6)]
        zd.append(pltpu.async_copy(
            stage.at[pl.ds(0, STAGE_TAIL)],
            acc.at[pl.ds(base + 6 * STAGE_SZ, STAGE_TAIL)], sem_z))

        @pl.when(s < SC_TILES)
        def _load():
            ld = [pltpu.async_copy(
                idx_hbm.at[pl.ds(s * ROWS_PER_TILE, ROWS_PER_TILE)], idx_v,
                sem_l),
                pltpu.async_copy(
                val_hbm.at[pl.ds(s * ROWS_PER_TILE, ROWS_PER_TILE)], val_v,
                sem_l)]
            for d in ld:
                d.wait()

        for d in zd:
            d.wait()
        plsc.subcore_barrier()

        @pl.when(s < SC_TILES)
        def _scatter():
            def body(t, carry):
                ds = [pltpu.async_copy(
                    val_v.at[t * WAVE + jj], acc.at[idx_v.at[t * WAVE + jj]],
                    sem_s, add=True) for jj in range(WAVE)]
                for d in ds:
                    d.wait()
                return carry

            lax.fori_loop(0, ROWS_PER_TILE // WAVE, body, jnp.int32(0))

        plsc.subcore_barrier()
        halves = [stage.at[pl.ds(0, DRAIN_CH)],
                  stage.at[pl.ds(DRAIN_CH, DRAIN_CH)]]
        sin = [sem_i0, sem_i1]
        sout = [sem_o0, sem_o1]
        ins = {}
        outs = {}

        def chunk(q):
            return pl.ds(base + q * DRAIN_CH, DRAIN_CH)

        ins[0] = pltpu.async_copy(acc.at[chunk(0)], halves[0], sin[0])
        ins[1] = pltpu.async_copy(acc.at[chunk(1)], halves[1], sin[1])
        for q in range(N_DRAIN):
            b = q % 2
            ins[q].wait()
            outs[q] = pltpu.async_copy(halves[b], out_hbm.at[chunk(q)],
                                       sout[b])
            if q + 2 < N_DRAIN:
                outs[q].wait()
                ins[q + 2] = pltpu.async_copy(acc.at[chunk(q + 2)],
                                              halves[b], sin[b])
        outs[N_DRAIN - 2].wait()
        outs[N_DRAIN - 1].wait()
        tail = pl.ds(base + N_DRAIN * DRAIN_CH, STAGE_TAIL)
        pltpu.async_copy(acc.at[tail], halves[0].at[pl.ds(0, STAGE_TAIL)],
                         sem_i0).wait()
        pltpu.async_copy(halves[0].at[pl.ds(0, STAGE_TAIL)],
                         out_hbm.at[tail], sem_o0).wait()


@functools.cache
def _sc_scatter_kernel():
    return pl.kernel(
        _sc_scatter_body,
        mesh=plsc.VectorSubcoreMesh(
            core_axis_name="c", subcore_axis_name="s", num_cores=2),
        out_type=jax.ShapeDtypeStruct((NOUT_PAD,), jnp.float32),
        scratch_types=[
            pltpu.VMEM((ROWS_PER_TILE, 128), jnp.int32),
            pltpu.VMEM((ROWS_PER_TILE, 128), jnp.float32),
            pltpu.VMEM((STAGE_SZ,), jnp.float32),
            pltpu.VMEM_SHARED((NOUT_PAD,), jnp.float32),
            pltpu.SemaphoreType.DMA,
            pltpu.SemaphoreType.DMA,
            pltpu.SemaphoreType.DMA,
            pltpu.SemaphoreType.DMA,
            pltpu.SemaphoreType.DMA,
            pltpu.SemaphoreType.DMA,
            pltpu.SemaphoreType.DMA,
        ],
    )


def kernel(inputs, layer_output, layer_weights):
    x = layer_output[0]
    xs = jnp.pad(x, ((0, 5), (0, 3), (0, 0))).reshape(116, 57, 2, CI)
    xs = xs.transpose(0, 3, 2, 1)
    wt = layer_weights.reshape(P, CO).T
    gidx = _tc_argmax(xs, wt)
    t = jnp.pad(inputs[0], ((0, 1), (0, 1), (0, 0)))
    vals = t.reshape(NG, 2, WOP, CO).transpose(0, 3, 1, 2)
    idx2d = gidx.reshape(-1, 128)
    val2d = vals.reshape(-1, 128)
    out = _sc_scatter_kernel()(idx2d, val2d)
    return out[:NOUT].reshape(1, HI, WI, CI)
